# Initial kernel scaffold; baseline (speedup 1.0000x reference)
import jax
import jax.numpy as jnp
from jax.experimental import pallas as pl


def kernel(x, W_in, b_in, g_in, be_in, W1, asrc1, adst1, b1, g1, be1, Wq, bq, Wk, bk, Wv, bv, Ws, bs_, g2, be2, W2, asrc2, adst2, b2, g3, be3, Wh1, bh1, Wh2, bh2, Wh3, bh3):
    raise NotImplementedError("write your pallas kernel here")


# speedup vs baseline: 3974.5812x; 3974.5812x over previous
import jax
import jax.numpy as jnp
from jax.experimental import pallas as pl
from jax.experimental.pallas import tpu as pltpu

_BS = 16384
_HID = 128
_OUT = 512
_TILE = 2048


def _ln_in(h, g, b):
    m = jnp.mean(h, axis=-1, keepdims=True)
    d = h - m
    v = jnp.mean(d * d, axis=-1, keepdims=True)
    return d * jax.lax.rsqrt(v + 1e-5) * g + b


def _gelu_in(v):
    return v * 0.5 * (1.0 + jax.lax.erf(v * 0.7071067811865475))


def _fused_kernel(x_ref, Win_ref, bin_ref, gin_ref, bein_ref,
                  W1_ref, b1_ref, g1_ref, be1_ref,
                  Wv_ref, Ws_ref, bvs_ref, g2_ref, be2_ref,
                  W2_ref, b2_ref, g3_ref, be3_ref,
                  Wh1_ref, bh1_ref, Wh2_ref, bh2_ref,
                  Wh3_ref, bh3_ref, o_ref):
    f32 = jnp.float32
    h = jnp.dot(x_ref[...], Win_ref[...], preferred_element_type=f32) + bin_ref[...]
    h = _ln_in(_gelu_in(h), gin_ref[...], bein_ref[...])
    h = jnp.dot(h, W1_ref[...], preferred_element_type=f32) + b1_ref[...]
    h = _ln_in(jnp.maximum(h, 0.0), g1_ref[...], be1_ref[...])
    h = jnp.dot(h, Wv_ref[...] + Ws_ref[...], preferred_element_type=f32) + bvs_ref[...]
    h = _ln_in(h, g2_ref[...], be2_ref[...])
    h = jnp.dot(h, W2_ref[...], preferred_element_type=f32) + b2_ref[...]
    h = _ln_in(h, g3_ref[...], be3_ref[...])
    z = _gelu_in(jnp.dot(h, Wh1_ref[...], preferred_element_type=f32) + bh1_ref[...])
    z = _gelu_in(jnp.dot(z, Wh2_ref[...], preferred_element_type=f32) + bh2_ref[...])
    o_ref[...] = jnp.dot(z, Wh3_ref[...], preferred_element_type=f32) + bh3_ref[...]


def _row(v):
    return v.reshape(1, -1)


def kernel(x, W_in, b_in, g_in, be_in, W1, asrc1, adst1, b1, g1, be1,
           Wq, bq, Wk, bk, Wv, bv, Ws, bs_, g2, be2,
           W2, asrc2, adst2, b2, g3, be3,
           Wh1, bh1, Wh2, bh2, Wh3, bh3):
    del asrc1, adst1, Wq, bq, Wk, bk, asrc2, adst2

    xp = jnp.pad(x, ((0, 0), (0, 2)))
    Winp = jnp.pad(W_in, ((0, 2), (0, 0)))
    bvs = bv + bs_

    grid = (_BS // _TILE,)

    def full(shape):
        nd = len(shape)
        return pl.BlockSpec(shape, lambda i: (0,) * nd)

    in_specs = [
        pl.BlockSpec((_TILE, 32), lambda i: (i, 0)),
        full((32, _HID)), full((1, _HID)), full((1, _HID)), full((1, _HID)),
        full((_HID, _HID)), full((1, _HID)), full((1, _HID)), full((1, _HID)),
        full((_HID, _HID)), full((_HID, _HID)), full((1, _HID)), full((1, _HID)), full((1, _HID)),
        full((_HID, _HID)), full((1, _HID)), full((1, _HID)), full((1, _HID)),
        full((_HID, _HID)), full((1, _HID)),
        full((_HID, _HID // 2)), full((1, _HID // 2)),
        full((_HID // 2, _OUT)), full((1, _OUT)),
    ]

    return pl.pallas_call(
        _fused_kernel,
        grid=grid,
        in_specs=in_specs,
        out_specs=pl.BlockSpec((_TILE, _OUT), lambda i: (i, 0)),
        out_shape=jax.ShapeDtypeStruct((_BS, _OUT), jnp.float32),
        compiler_params=pltpu.CompilerParams(
            dimension_semantics=("parallel",),
        ),
    )(xp, Winp, _row(b_in), _row(g_in), _row(be_in),
      W1, _row(b1), _row(g1), _row(be1),
      Wv, Ws, _row(bvs), _row(g2), _row(be2),
      W2, _row(b2), _row(g3), _row(be3),
      Wh1, _row(bh1), Wh2, _row(bh2),
      Wh3, _row(bh3))
